# baseline (speedup 1.0000x reference)
import functools

import jax
import jax.numpy as jnp
import numpy as np
from jax import lax
from jax.experimental import pallas as pl
from jax.experimental.pallas import tpu as pltpu
from jax.experimental.pallas import tpu_sc as plsc

_N = 50000
_E = 800000
_G = 128
_ND = 7
_ED = 4
_H = 128
_OUT = 128
_TDA_DIM = 32
_TDA_PROJ = 64
_FUSE = _OUT + _TDA_PROJ
_K = 4
_NLAYERS = 4

_BN = 2000
_BE = 4000


def _nan_clean(v, posinf, neginf):
    return jnp.nan_to_num(v, nan=0.0, posinf=posinf, neginf=neginf)


def _in_body(x_ref, w_ref, b_ref, h_ref):
    xv = _nan_clean(x_ref[...], 3.0, -3.0)
    u = jnp.dot(xv, w_ref[...], preferred_element_type=jnp.float32) + b_ref[...]
    h_ref[...] = u * jax.nn.sigmoid(u)


def _input_proj(xp, w8, b):
    nb = _N // _BN
    return pl.pallas_call(
        _in_body,
        grid=(nb,),
        in_specs=[
            pl.BlockSpec((_BN, 8), lambda i: (i, 0)),
            pl.BlockSpec((8, _H), lambda i: (0, 0)),
            pl.BlockSpec((1, _H), lambda i: (0, 0)),
        ],
        out_specs=pl.BlockSpec((_BN, _H), lambda i: (i, 0)),
        out_shape=jax.ShapeDtypeStruct((_N, _H), jnp.float32),
    )(xp, w8, b)


def _prep_body(h_ref, wd_ref, ws_ref, d_ref, s_ref):
    h = h_ref[...]
    d_ref[...] = jnp.dot(h, wd_ref[...], preferred_element_type=jnp.float32)
    s_ref[...] = jnp.dot(h, ws_ref[...], preferred_element_type=jnp.float32)


def _node_prep(h, wd, ws):
    nb = _N // _BN
    return pl.pallas_call(
        _prep_body,
        grid=(nb,),
        in_specs=[
            pl.BlockSpec((_BN, _H), lambda i: (i, 0)),
            pl.BlockSpec((_H, 2 * _H), lambda i: (0, 0)),
            pl.BlockSpec((_H, 2 * _H), lambda i: (0, 0)),
        ],
        out_specs=[
            pl.BlockSpec((_BN, 2 * _H), lambda i: (i, 0)),
            pl.BlockSpec((_BN, 2 * _H), lambda i: (i, 0)),
        ],
        out_shape=[
            jax.ShapeDtypeStruct((_N, 2 * _H), jnp.float32),
            jax.ShapeDtypeStruct((_N, 2 * _H), jnp.float32),
        ],
    )(h, wd, ws)


_NW = 32
_EH1 = 384000
_BG = 40
_SBG = 1000
_RING = 4


def _sc_gather_pair(d_mat, s_mat, dst, src, n_edges):
    epw = n_edges // _NW
    mesh = plsc.VectorSubcoreMesh(core_axis_name="c", subcore_axis_name="s")

    @functools.partial(
        pl.kernel,
        mesh=mesh,
        out_type=[
            jax.ShapeDtypeStruct((n_edges, 2 * _H), jnp.float32),
            jax.ShapeDtypeStruct((n_edges, 2 * _H), jnp.float32),
        ],
        scratch_types=[
            pltpu.VMEM((_SBG,), jnp.int32),
            pltpu.VMEM((_SBG,), jnp.int32),
            pltpu.VMEM((_RING, _BG, 2 * _H), jnp.float32),
            pltpu.VMEM((_RING, _BG, 2 * _H), jnp.float32),
            pltpu.SemaphoreType.DMA,
            pltpu.SemaphoreType.DMA,
            pltpu.SemaphoreType.DMA,
            pltpu.SemaphoreType.DMA,
        ],
    )
    def k(d_hbm, s_hbm, dst_hbm, src_hbm, dg_hbm, sg_hbm,
          idxd, idxs, dbufs, sbufs, gsd, gss, wsd, wss):
        wid = lax.axis_index("s") * 2 + lax.axis_index("c")
        base = wid * epw

        def super_body(sb, carry):
            soff = base + sb * _SBG
            pltpu.sync_copy(dst_hbm.at[pl.ds(soff, _SBG)], idxd)
            pltpu.sync_copy(src_hbm.at[pl.ds(soff, _SBG)], idxs)
            g = [None] * _RING
            w = [None] * _RING

            def issue_gather(j):
                r = j % _RING
                c1 = pltpu.async_copy(
                    d_hbm.at[idxd.at[pl.ds(j * _BG, _BG)]], dbufs.at[r], gsd)
                c2 = pltpu.async_copy(
                    s_hbm.at[idxs.at[pl.ds(j * _BG, _BG)]], sbufs.at[r], gss)
                return (c1, c2)

            def issue_wb(j):
                r = j % _RING
                g[r][0].wait()
                g[r][1].wait()
                off = soff + j * _BG
                c1 = pltpu.async_copy(dbufs.at[r], dg_hbm.at[pl.ds(off, _BG)], wsd)
                c2 = pltpu.async_copy(sbufs.at[r], sg_hbm.at[pl.ds(off, _BG)], wss)
                return (c1, c2)

            nblk = _SBG // _BG
            for j in range(nblk):
                r = j % _RING
                if w[r] is not None:
                    w[r][0].wait()
                    w[r][1].wait()
                    w[r] = None
                g[r] = issue_gather(j)
                jw = j - (_RING - 1)
                if jw >= 0:
                    w[jw % _RING] = issue_wb(jw)
            for jw in range(nblk - (_RING - 1), nblk):
                w[jw % _RING] = issue_wb(jw)
            for r in range(_RING):
                if w[r] is not None:
                    w[r][0].wait()
                    w[r][1].wait()
            return carry

        lax.fori_loop(0, epw // _SBG, super_body, 0)

    return k(d_mat, s_mat, dst, src)


_NCK = 12544
_NPAD = 4 * _NCK
_BS = 80
_EPT = _E // 16
_RPT = _NCK // 16


_SBS = 2000


def _sc_scatter_add(m1, m2, dst1, dst2, zrows):
    n1, n2 = m1.shape[0], m2.shape[0]
    mesh = plsc.VectorSubcoreMesh(core_axis_name="c", subcore_axis_name="s")

    @functools.partial(
        pl.kernel,
        mesh=mesh,
        out_type=jax.ShapeDtypeStruct((_NPAD, _H), jnp.float32),
        scratch_types=[
            pltpu.VMEM((_SBS,), jnp.int32),
            pltpu.VMEM((_BS,), jnp.int32),
            pltpu.VMEM((2, _BS, _H), jnp.float32),
            pltpu.VMEM_SHARED((_NCK + 16, _H), jnp.float32),
            pltpu.SemaphoreType.DMA,
        ],
    )
    def k(m1_hbm, m2_hbm, dst1_hbm, dst2_hbm, z_hbm, agg_hbm,
          dstb, idxb, mbufs, shard, lsem):
        cid = lax.axis_index("c")
        sid = lax.axis_index("s")
        nblk = _SBS // _BS
        for rnd in range(2):
            chunk = rnd * 2 + cid
            nbase = chunk * _NCK
            dummy = _NCK + sid

            @pl.when(sid == 0)
            def _():
                pltpu.sync_copy(z_hbm, shard)

            plsc.subcore_barrier()

            for m_hbm, dst_hbm, n_e in ((m1_hbm, dst1_hbm, n1),
                                        (m2_hbm, dst2_hbm, n2)):
                ept = n_e // 16
                ebase = sid * ept

                def super_body(sb, carry):
                    soff = ebase + sb * _SBS
                    pltpu.sync_copy(dst_hbm.at[pl.ds(soff, _SBS)], dstb)
                    L = [None, None]
                    L[0] = pltpu.async_copy(
                        m_hbm.at[pl.ds(soff, _BS)], mbufs.at[0], lsem)
                    L[1] = pltpu.async_copy(
                        m_hbm.at[pl.ds(soff + _BS, _BS)], mbufs.at[1], lsem)
                    for j in range(nblk):
                        r = j % 2
                        L[r].wait()
                        for v in range(_BS // 16):
                            d = dstb[pl.ds(j * _BS + v * 16, 16)]
                            loc = d - nbase
                            ok = (loc >= 0) & (loc < _NCK)
                            idxb[pl.ds(v * 16, 16)] = jnp.where(ok, loc, dummy)
                        pltpu.sync_copy(mbufs.at[r], shard.at[idxb], add=True)
                        if j + 2 < nblk:
                            L[r] = pltpu.async_copy(
                                m_hbm.at[pl.ds(soff + (j + 2) * _BS, _BS)],
                                mbufs.at[r], lsem)
                    return carry

                lax.fori_loop(0, ept // _SBS, super_body, 0)
            plsc.subcore_barrier()
            pltpu.sync_copy(shard.at[pl.ds(sid * _RPT, _RPT)],
                            agg_hbm.at[pl.ds(nbase + sid * _RPT, _RPT)])
            plsc.subcore_barrier()

    return k(m1, m2, dst1, dst2, zrows)


def _edge_body(dg_ref, sg_ref, ea_ref, wfe_ref, bf_ref, wse_ref, bs_ref, m_ref):
    dg = dg_ref[...]
    sg = sg_ref[...]
    ea = ea_ref[...]
    uf = (dg[:, :_H] + sg[:, :_H]
          + jnp.dot(ea, wfe_ref[...], preferred_element_type=jnp.float32)
          + bf_ref[...])
    us = (dg[:, _H:] + sg[:, _H:]
          + jnp.dot(ea, wse_ref[...], preferred_element_type=jnp.float32)
          + bs_ref[...])
    sig = jax.nn.sigmoid(uf)
    sp = jnp.maximum(us, 0.0) + jnp.log1p(jnp.exp(-jnp.abs(us)))
    m_ref[...] = sig * sp


def _edge_stage(dg, sg, ea8, wfe8, bf, wse8, bs):
    nb = dg.shape[0] // _BE
    return pl.pallas_call(
        _edge_body,
        grid=(nb,),
        in_specs=[
            pl.BlockSpec((_BE, 2 * _H), lambda i: (i, 0)),
            pl.BlockSpec((_BE, 2 * _H), lambda i: (i, 0)),
            pl.BlockSpec((_BE, 8), lambda i: (i, 0)),
            pl.BlockSpec((8, _H), lambda i: (0, 0)),
            pl.BlockSpec((1, _H), lambda i: (0, 0)),
            pl.BlockSpec((8, _H), lambda i: (0, 0)),
            pl.BlockSpec((1, _H), lambda i: (0, 0)),
        ],
        out_specs=pl.BlockSpec((_BE, _H), lambda i: (i, 0)),
        out_shape=jax.ShapeDtypeStruct((dg.shape[0], _H), jnp.float32),
    )(dg, sg, ea8, wfe8, bf, wse8, bs)


def _upd_body(h_ref, agg_ref, g_ref, b_ref, out_ref):
    h = h_ref[...]
    c = agg_ref[...] + h
    y = c * jax.nn.sigmoid(c) + h
    m = jnp.mean(y, axis=-1, keepdims=True)
    v = jnp.mean((y - m) ** 2, axis=-1, keepdims=True)
    out_ref[...] = (y - m) * jax.lax.rsqrt(v + 1e-5) * g_ref[...] + b_ref[...]


def _update(h, agg, g, b):
    nb = _N // _BN
    return pl.pallas_call(
        _upd_body,
        grid=(nb,),
        in_specs=[
            pl.BlockSpec((_BN, _H), lambda i: (i, 0)),
            pl.BlockSpec((_BN, _H), lambda i: (i, 0)),
            pl.BlockSpec((1, _H), lambda i: (0, 0)),
            pl.BlockSpec((1, _H), lambda i: (0, 0)),
        ],
        out_specs=pl.BlockSpec((_BN, _H), lambda i: (i, 0)),
        out_shape=jax.ShapeDtypeStruct((_N, _H), jnp.float32),
    )(h, agg, g, b)


def _pool_body(h_ref, batch_ref, sums_ref, cnt_ref):
    i = pl.program_id(0)

    @pl.when(i == 0)
    def _():
        sums_ref[...] = jnp.zeros_like(sums_ref)
        cnt_ref[...] = jnp.zeros_like(cnt_ref)

    b = batch_ref[0, 0, :]
    onehot = (b[:, None] == jax.lax.broadcasted_iota(jnp.int32, (_BN, _G), 1))
    onehot = onehot.astype(jnp.float32)
    sums_ref[...] += jax.lax.dot_general(
        onehot, h_ref[...], (((0,), (0,)), ((), ())),
        preferred_element_type=jnp.float32)
    cnt_ref[...] += jnp.sum(onehot, axis=0, keepdims=True)


def _pool(h, batch3):
    nb = _N // _BN
    return pl.pallas_call(
        _pool_body,
        grid=(nb,),
        in_specs=[
            pl.BlockSpec((_BN, _H), lambda i: (i, 0)),
            pl.BlockSpec((1, 1, _BN), lambda i: (i, 0, 0)),
        ],
        out_specs=[
            pl.BlockSpec((_G, _H), lambda i: (0, 0)),
            pl.BlockSpec((1, _G), lambda i: (0, 0)),
        ],
        out_shape=[
            jax.ShapeDtypeStruct((_G, _H), jnp.float32),
            jax.ShapeDtypeStruct((1, _G), jnp.float32),
        ],
    )(h, batch3)


def _ln_rows(x, g, b):
    m = jnp.mean(x, axis=-1, keepdims=True)
    v = jnp.mean((x - m) ** 2, axis=-1, keepdims=True)
    return (x - m) * jax.lax.rsqrt(v + 1e-5) * g + b


def _head_body(sums_ref, cnt_ref, tda_ref,
               outw_ref, outb_ref, tw1_ref, tb1_ref, tg_ref, tbn_ref,
               tw2_ref, tb2_ref, fg_ref, fb_ref,
               g1w_ref, g1b_ref, g2w_ref, g2b_ref,
               lw_ref, lb_ref, q1w_ref, q1b_ref, q2w_ref, q2b_ref,
               yhat_ref, zf_ref):
    cnt = jnp.maximum(cnt_ref[...], 1.0)
    pooled = sums_ref[...] / cnt.reshape(_G, 1)
    z_gnn = jnp.dot(pooled, outw_ref[...],
                    preferred_element_type=jnp.float32) + outb_ref[...]
    tda = _nan_clean(tda_ref[...], 3.0, -3.0)
    t = jnp.dot(tda, tw1_ref[...], preferred_element_type=jnp.float32) + tb1_ref[...]
    t = t * jax.nn.sigmoid(t)
    t = _ln_rows(t, tg_ref[...], tbn_ref[...])
    z_tda = jnp.dot(t, tw2_ref[...], preferred_element_type=jnp.float32) + tb2_ref[...]
    z_gnn = jnp.nan_to_num(z_gnn, nan=0.0)
    z_tda = jnp.nan_to_num(z_tda, nan=0.0)
    zf = jnp.concatenate([z_gnn, z_tda], axis=-1)
    zf = _ln_rows(zf, fg_ref[...], fb_ref[...])
    zf_ref[...] = zf
    gh = jnp.dot(zf, g1w_ref[...], preferred_element_type=jnp.float32) + g1b_ref[...]
    gh = gh * jax.nn.sigmoid(gh)
    glog = jnp.dot(gh, g2w_ref[...], preferred_element_type=jnp.float32) + g2b_ref[...]
    gates = jax.nn.softmax(glog, axis=-1)
    lin = jnp.dot(zf, lw_ref[...], preferred_element_type=jnp.float32) + lb_ref[...]
    q = jnp.dot(zf, q1w_ref[...], preferred_element_type=jnp.float32) + q1b_ref[...]
    q = q * jax.nn.sigmoid(q)
    quad = jnp.dot(q, q2w_ref[...], preferred_element_type=jnp.float32) + q2b_ref[...]
    preds = lin + quad
    yhat_ref[...] = jnp.sum(gates * preds, axis=-1, keepdims=True)


def _heads(sums, cnt, tda, wpack):
    specs = [
        pl.BlockSpec((_G, _H), lambda: (0, 0)),
        pl.BlockSpec((1, _G), lambda: (0, 0)),
        pl.BlockSpec((_G, _TDA_DIM), lambda: (0, 0)),
    ]
    args = [sums, cnt, tda]
    for w in wpack:
        args.append(w)
        specs.append(pl.BlockSpec(w.shape, lambda: (0,) * w.ndim))
    return pl.pallas_call(
        _head_body,
        in_specs=specs,
        out_specs=[
            pl.BlockSpec((_G, 1), lambda: (0, 0)),
            pl.BlockSpec((_G, _FUSE), lambda: (0, 0)),
        ],
        out_shape=[
            jax.ShapeDtypeStruct((_G, 1), jnp.float32),
            jax.ShapeDtypeStruct((_G, _FUSE), jnp.float32),
        ],
    )(*args)


def kernel(x, edge_index, edge_attr, batch, tda, params):
    p = params
    xp = jnp.pad(x, ((0, 0), (0, 1)))
    w8 = jnp.pad(p["in_W"], ((0, 1), (0, 0)))
    h = _input_proj(xp, w8, p["in_b"].reshape(1, _H))

    src = edge_index[0]
    dst = edge_index[1]
    src1, src2 = src[:_EH1], src[_EH1:]
    dst1, dst2 = dst[:_EH1], dst[_EH1:]
    ea = _nan_clean(edge_attr, 1.0, 0.0)
    ea8 = jnp.pad(ea, ((0, 0), (0, 8 - _ED)))
    ea8_1, ea8_2 = ea8[:_EH1], ea8[_EH1:]
    zrows = jnp.zeros((_NCK + 16, _H), jnp.float32)

    for conv, ln in zip(p["convs"], p["lns"]):
        wf, ws = conv["Wf"], conv["Ws"]
        wd = jnp.concatenate([wf[:_H], ws[:_H]], axis=1)
        wsrc = jnp.concatenate([wf[_H:2 * _H], ws[_H:2 * _H]], axis=1)
        wfe8 = jnp.pad(wf[2 * _H:], ((0, 8 - _ED), (0, 0)))
        wse8 = jnp.pad(ws[2 * _H:], ((0, 8 - _ED), (0, 0)))
        d_mat, s_mat = _node_prep(h, wd, wsrc)
        dg1, sg1 = _sc_gather_pair(d_mat, s_mat, dst1, src1, _EH1)
        dg2, sg2 = _sc_gather_pair(d_mat, s_mat, dst2, src2, _E - _EH1)
        bf = conv["bf"].reshape(1, _H)
        bs = conv["bs"].reshape(1, _H)
        m1 = _edge_stage(dg1, sg1, ea8_1, wfe8, bf, wse8, bs)
        m2 = _edge_stage(dg2, sg2, ea8_2, wfe8, bf, wse8, bs)
        agg = _sc_scatter_add(m1, m2, dst1, dst2, zrows)[:_N]
        h = _update(h, agg, ln["g"].reshape(1, _H), ln["b"].reshape(1, _H))

    batch3 = batch.reshape(_N // _BN, 1, _BN)
    sums, cnt = _pool(h, batch3)

    q1w = jnp.concatenate([pk["q1W"] for pk in p["poly"]], axis=1)
    q1b = jnp.concatenate([pk["q1b"] for pk in p["poly"]], axis=0).reshape(1, -1)
    hq = _FUSE // 2
    q2w = jnp.zeros((_K * hq, _K), jnp.float32)
    for k in range(_K):
        q2w = q2w.at[k * hq:(k + 1) * hq, k].set(p["poly"][k]["q2W"][:, 0])
    q2b = jnp.concatenate([pk["q2b"] for pk in p["poly"]], axis=0).reshape(1, _K)
    lw = jnp.concatenate([pk["lW"] for pk in p["poly"]], axis=1)
    lb = jnp.concatenate([pk["lb"] for pk in p["poly"]], axis=0).reshape(1, _K)

    wpack = [
        p["out_W"], p["out_b"].reshape(1, _OUT),
        p["tda_W1"], p["tda_b1"].reshape(1, 2 * _TDA_PROJ),
        p["tda_g"].reshape(1, 2 * _TDA_PROJ), p["tda_bn"].reshape(1, 2 * _TDA_PROJ),
        p["tda_W2"], p["tda_b2"].reshape(1, _TDA_PROJ),
        p["fuse_g"].reshape(1, _FUSE), p["fuse_b"].reshape(1, _FUSE),
        p["g1W"], p["g1b"].reshape(1, _K * 4),
        p["g2W"], p["g2b"].reshape(1, _K),
        lw, lb, q1w, q1b, q2w, q2b,
    ]
    yhat2, zf = _heads(sums, cnt, tda, wpack)
    return yhat2.reshape(_G), zf


# speedup vs baseline: 1.0906x; 1.0906x over previous
import functools

import jax
import jax.numpy as jnp
import numpy as np
from jax import lax
from jax.experimental import pallas as pl
from jax.experimental.pallas import tpu as pltpu
from jax.experimental.pallas import tpu_sc as plsc

_N = 50000
_E = 800000
_G = 128
_ND = 7
_ED = 4
_H = 128
_OUT = 128
_TDA_DIM = 32
_TDA_PROJ = 64
_FUSE = _OUT + _TDA_PROJ
_K = 4
_NLAYERS = 4

_BN = 2000
_BE = 4000


def _nan_clean(v, posinf, neginf):
    return jnp.nan_to_num(v, nan=0.0, posinf=posinf, neginf=neginf)


def _in_body(x_ref, w_ref, b_ref, h_ref):
    xv = _nan_clean(x_ref[...], 3.0, -3.0)
    u = jnp.dot(xv, w_ref[...], preferred_element_type=jnp.float32) + b_ref[...]
    h_ref[...] = u * jax.nn.sigmoid(u)


def _input_proj(xp, w8, b):
    nb = _N // _BN
    return pl.pallas_call(
        _in_body,
        grid=(nb,),
        in_specs=[
            pl.BlockSpec((_BN, 8), lambda i: (i, 0)),
            pl.BlockSpec((8, _H), lambda i: (0, 0)),
            pl.BlockSpec((1, _H), lambda i: (0, 0)),
        ],
        out_specs=pl.BlockSpec((_BN, _H), lambda i: (i, 0)),
        out_shape=jax.ShapeDtypeStruct((_N, _H), jnp.float32),
    )(xp, w8, b)


def _prep_body(h_ref, wd_ref, ws_ref, d_ref, s_ref):
    h = h_ref[...]
    d_ref[...] = jnp.dot(h, wd_ref[...], preferred_element_type=jnp.float32)
    s_ref[...] = jnp.dot(h, ws_ref[...], preferred_element_type=jnp.float32)


def _node_prep(h, wd, ws):
    nb = _N // _BN
    return pl.pallas_call(
        _prep_body,
        grid=(nb,),
        in_specs=[
            pl.BlockSpec((_BN, _H), lambda i: (i, 0)),
            pl.BlockSpec((_H, 2 * _H), lambda i: (0, 0)),
            pl.BlockSpec((_H, 2 * _H), lambda i: (0, 0)),
        ],
        out_specs=[
            pl.BlockSpec((_BN, 2 * _H), lambda i: (i, 0)),
            pl.BlockSpec((_BN, 2 * _H), lambda i: (i, 0)),
        ],
        out_shape=[
            jax.ShapeDtypeStruct((_N, 2 * _H), jnp.float32),
            jax.ShapeDtypeStruct((_N, 2 * _H), jnp.float32),
        ],
    )(h, wd, ws)


_NW = 32
_EH1 = 384000
_BG = 40
_SBG = 1000
_RING = 4


def _sc_gather_pair(d_mat, s_mat, dst, src, n_edges):
    epw = n_edges // _NW
    mesh = plsc.VectorSubcoreMesh(core_axis_name="c", subcore_axis_name="s")

    @functools.partial(
        pl.kernel,
        mesh=mesh,
        out_type=jax.ShapeDtypeStruct((n_edges, 2 * _H), jnp.float32),
        scratch_types=[
            pltpu.VMEM((_SBG,), jnp.int32),
            pltpu.VMEM((_SBG,), jnp.int32),
            pltpu.VMEM((_RING, _BG, 2 * _H), jnp.float32),
            pltpu.VMEM((_RING, _BG, 2 * _H), jnp.float32),
            pltpu.SemaphoreType.DMA,
            pltpu.SemaphoreType.DMA,
            pltpu.SemaphoreType.DMA,
            pltpu.SemaphoreType.DMA,
        ],
    )
    def k(d_hbm, s_hbm, dst_hbm, src_hbm, u_hbm,
          idxd, idxs, dbufs, sbufs, gsd, gss, wsd, wss):
        wid = lax.axis_index("s") * 2 + lax.axis_index("c")
        base = wid * epw

        def super_body(sb, carry):
            soff = base + sb * _SBG
            pltpu.sync_copy(dst_hbm.at[pl.ds(soff, _SBG)], idxd)
            pltpu.sync_copy(src_hbm.at[pl.ds(soff, _SBG)], idxs)
            g = [None] * _RING
            w = [None] * _RING

            def issue_gather(j):
                r = j % _RING
                c1 = pltpu.async_copy(
                    d_hbm.at[idxd.at[pl.ds(j * _BG, _BG)]], dbufs.at[r], gsd)
                c2 = pltpu.async_copy(
                    s_hbm.at[idxs.at[pl.ds(j * _BG, _BG)]], sbufs.at[r], gss)
                return (c1, c2)

            def issue_wb(j):
                r = j % _RING

                def add_row(i, carry):
                    for v in range(2 * _H // 16):
                        sl = pl.ds(v * 16, 16)
                        dbufs[r, i, sl] = dbufs[r, i, sl] + sbufs[r, i, sl]
                    return carry

                g[r][0].wait()
                g[r][1].wait()
                lax.fori_loop(0, _BG, add_row, 0)
                off = soff + j * _BG
                c1 = pltpu.async_copy(dbufs.at[r], u_hbm.at[pl.ds(off, _BG)], wsd)
                return (c1,)

            nblk = _SBG // _BG
            for j in range(nblk):
                r = j % _RING
                if w[r] is not None:
                    w[r][0].wait()
                    w[r] = None
                g[r] = issue_gather(j)
                jw = j - (_RING - 1)
                if jw >= 0:
                    w[jw % _RING] = issue_wb(jw)
            for jw in range(nblk - (_RING - 1), nblk):
                w[jw % _RING] = issue_wb(jw)
            for r in range(_RING):
                if w[r] is not None:
                    w[r][0].wait()
            return carry

        lax.fori_loop(0, epw // _SBG, super_body, 0)

    return k(d_mat, s_mat, dst, src)


_NCK = 12544
_NPAD = 4 * _NCK
_BS = 80
_EPT = _E // 16
_RPT = _NCK // 16


_SBS = 2000


def _sc_scatter_add(m1, m2, dst1, dst2, zrows):
    n1, n2 = m1.shape[0], m2.shape[0]
    mesh = plsc.VectorSubcoreMesh(core_axis_name="c", subcore_axis_name="s")

    @functools.partial(
        pl.kernel,
        mesh=mesh,
        out_type=jax.ShapeDtypeStruct((_NPAD, _H), jnp.float32),
        scratch_types=[
            pltpu.VMEM((_SBS,), jnp.int32),
            pltpu.VMEM((_BS,), jnp.int32),
            pltpu.VMEM((2, _BS, _H), jnp.float32),
            pltpu.VMEM_SHARED((_NCK + 16, _H), jnp.float32),
            pltpu.SemaphoreType.DMA,
        ],
    )
    def k(m1_hbm, m2_hbm, dst1_hbm, dst2_hbm, z_hbm, agg_hbm,
          dstb, idxb, mbufs, shard, lsem):
        cid = lax.axis_index("c")
        sid = lax.axis_index("s")
        nblk = _SBS // _BS
        for rnd in range(2):
            chunk = rnd * 2 + cid
            nbase = chunk * _NCK
            dummy = _NCK + sid

            @pl.when(sid == 0)
            def _():
                pltpu.sync_copy(z_hbm, shard)

            plsc.subcore_barrier()

            for m_hbm, dst_hbm, n_e in ((m1_hbm, dst1_hbm, n1),
                                        (m2_hbm, dst2_hbm, n2)):
                ept = n_e // 16
                ebase = sid * ept

                def super_body(sb, carry):
                    soff = ebase + sb * _SBS
                    pltpu.sync_copy(dst_hbm.at[pl.ds(soff, _SBS)], dstb)
                    L = [None, None]
                    L[0] = pltpu.async_copy(
                        m_hbm.at[pl.ds(soff, _BS)], mbufs.at[0], lsem)
                    L[1] = pltpu.async_copy(
                        m_hbm.at[pl.ds(soff + _BS, _BS)], mbufs.at[1], lsem)
                    for j in range(nblk):
                        r = j % 2
                        L[r].wait()
                        for v in range(_BS // 16):
                            d = dstb[pl.ds(j * _BS + v * 16, 16)]
                            loc = d - nbase
                            ok = (loc >= 0) & (loc < _NCK)
                            idxb[pl.ds(v * 16, 16)] = jnp.where(ok, loc, dummy)
                        pltpu.sync_copy(mbufs.at[r], shard.at[idxb], add=True)
                        if j + 2 < nblk:
                            L[r] = pltpu.async_copy(
                                m_hbm.at[pl.ds(soff + (j + 2) * _BS, _BS)],
                                mbufs.at[r], lsem)
                    return carry

                lax.fori_loop(0, ept // _SBS, super_body, 0)
            plsc.subcore_barrier()
            pltpu.sync_copy(shard.at[pl.ds(sid * _RPT, _RPT)],
                            agg_hbm.at[pl.ds(nbase + sid * _RPT, _RPT)])
            plsc.subcore_barrier()

    return k(m1, m2, dst1, dst2, zrows)


def _edge_body(u_ref, ea_ref, wfe_ref, bf_ref, wse_ref, bs_ref, m_ref):
    u = u_ref[...]
    ea = ea_ref[...]
    uf = (u[:, :_H]
          + jnp.dot(ea, wfe_ref[...], preferred_element_type=jnp.float32)
          + bf_ref[...])
    us = (u[:, _H:]
          + jnp.dot(ea, wse_ref[...], preferred_element_type=jnp.float32)
          + bs_ref[...])
    sig = jax.nn.sigmoid(uf)
    sp = jnp.maximum(us, 0.0) + jnp.log1p(jnp.exp(-jnp.abs(us)))
    m_ref[...] = sig * sp


def _edge_stage(u, ea8, wfe8, bf, wse8, bs):
    nb = u.shape[0] // _BE
    return pl.pallas_call(
        _edge_body,
        grid=(nb,),
        in_specs=[
            pl.BlockSpec((_BE, 2 * _H), lambda i: (i, 0)),
            pl.BlockSpec((_BE, 8), lambda i: (i, 0)),
            pl.BlockSpec((8, _H), lambda i: (0, 0)),
            pl.BlockSpec((1, _H), lambda i: (0, 0)),
            pl.BlockSpec((8, _H), lambda i: (0, 0)),
            pl.BlockSpec((1, _H), lambda i: (0, 0)),
        ],
        out_specs=pl.BlockSpec((_BE, _H), lambda i: (i, 0)),
        out_shape=jax.ShapeDtypeStruct((u.shape[0], _H), jnp.float32),
    )(u, ea8, wfe8, bf, wse8, bs)


def _upd_body(h_ref, agg_ref, g_ref, b_ref, out_ref):
    h = h_ref[...]
    c = agg_ref[...] + h
    y = c * jax.nn.sigmoid(c) + h
    m = jnp.mean(y, axis=-1, keepdims=True)
    v = jnp.mean((y - m) ** 2, axis=-1, keepdims=True)
    out_ref[...] = (y - m) * jax.lax.rsqrt(v + 1e-5) * g_ref[...] + b_ref[...]


def _update(h, agg, g, b):
    nb = _N // _BN
    return pl.pallas_call(
        _upd_body,
        grid=(nb,),
        in_specs=[
            pl.BlockSpec((_BN, _H), lambda i: (i, 0)),
            pl.BlockSpec((_BN, _H), lambda i: (i, 0)),
            pl.BlockSpec((1, _H), lambda i: (0, 0)),
            pl.BlockSpec((1, _H), lambda i: (0, 0)),
        ],
        out_specs=pl.BlockSpec((_BN, _H), lambda i: (i, 0)),
        out_shape=jax.ShapeDtypeStruct((_N, _H), jnp.float32),
    )(h, agg, g, b)


def _pool_body(h_ref, batch_ref, sums_ref, cnt_ref):
    i = pl.program_id(0)

    @pl.when(i == 0)
    def _():
        sums_ref[...] = jnp.zeros_like(sums_ref)
        cnt_ref[...] = jnp.zeros_like(cnt_ref)

    b = batch_ref[0, 0, :]
    onehot = (b[:, None] == jax.lax.broadcasted_iota(jnp.int32, (_BN, _G), 1))
    onehot = onehot.astype(jnp.float32)
    sums_ref[...] += jax.lax.dot_general(
        onehot, h_ref[...], (((0,), (0,)), ((), ())),
        preferred_element_type=jnp.float32)
    cnt_ref[...] += jnp.sum(onehot, axis=0, keepdims=True)


def _pool(h, batch3):
    nb = _N // _BN
    return pl.pallas_call(
        _pool_body,
        grid=(nb,),
        in_specs=[
            pl.BlockSpec((_BN, _H), lambda i: (i, 0)),
            pl.BlockSpec((1, 1, _BN), lambda i: (i, 0, 0)),
        ],
        out_specs=[
            pl.BlockSpec((_G, _H), lambda i: (0, 0)),
            pl.BlockSpec((1, _G), lambda i: (0, 0)),
        ],
        out_shape=[
            jax.ShapeDtypeStruct((_G, _H), jnp.float32),
            jax.ShapeDtypeStruct((1, _G), jnp.float32),
        ],
    )(h, batch3)


def _ln_rows(x, g, b):
    m = jnp.mean(x, axis=-1, keepdims=True)
    v = jnp.mean((x - m) ** 2, axis=-1, keepdims=True)
    return (x - m) * jax.lax.rsqrt(v + 1e-5) * g + b


def _head_body(sums_ref, cnt_ref, tda_ref,
               outw_ref, outb_ref, tw1_ref, tb1_ref, tg_ref, tbn_ref,
               tw2_ref, tb2_ref, fg_ref, fb_ref,
               g1w_ref, g1b_ref, g2w_ref, g2b_ref,
               lw_ref, lb_ref, q1w_ref, q1b_ref, q2w_ref, q2b_ref,
               yhat_ref, zf_ref):
    cnt = jnp.maximum(cnt_ref[...], 1.0)
    pooled = sums_ref[...] / cnt.reshape(_G, 1)
    z_gnn = jnp.dot(pooled, outw_ref[...],
                    preferred_element_type=jnp.float32) + outb_ref[...]
    tda = _nan_clean(tda_ref[...], 3.0, -3.0)
    t = jnp.dot(tda, tw1_ref[...], preferred_element_type=jnp.float32) + tb1_ref[...]
    t = t * jax.nn.sigmoid(t)
    t = _ln_rows(t, tg_ref[...], tbn_ref[...])
    z_tda = jnp.dot(t, tw2_ref[...], preferred_element_type=jnp.float32) + tb2_ref[...]
    z_gnn = jnp.nan_to_num(z_gnn, nan=0.0)
    z_tda = jnp.nan_to_num(z_tda, nan=0.0)
    zf = jnp.concatenate([z_gnn, z_tda], axis=-1)
    zf = _ln_rows(zf, fg_ref[...], fb_ref[...])
    zf_ref[...] = zf
    gh = jnp.dot(zf, g1w_ref[...], preferred_element_type=jnp.float32) + g1b_ref[...]
    gh = gh * jax.nn.sigmoid(gh)
    glog = jnp.dot(gh, g2w_ref[...], preferred_element_type=jnp.float32) + g2b_ref[...]
    gates = jax.nn.softmax(glog, axis=-1)
    lin = jnp.dot(zf, lw_ref[...], preferred_element_type=jnp.float32) + lb_ref[...]
    q = jnp.dot(zf, q1w_ref[...], preferred_element_type=jnp.float32) + q1b_ref[...]
    q = q * jax.nn.sigmoid(q)
    quad = jnp.dot(q, q2w_ref[...], preferred_element_type=jnp.float32) + q2b_ref[...]
    preds = lin + quad
    yhat_ref[...] = jnp.sum(gates * preds, axis=-1, keepdims=True)


def _heads(sums, cnt, tda, wpack):
    specs = [
        pl.BlockSpec((_G, _H), lambda: (0, 0)),
        pl.BlockSpec((1, _G), lambda: (0, 0)),
        pl.BlockSpec((_G, _TDA_DIM), lambda: (0, 0)),
    ]
    args = [sums, cnt, tda]
    for w in wpack:
        args.append(w)
        specs.append(pl.BlockSpec(w.shape, lambda: (0,) * w.ndim))
    return pl.pallas_call(
        _head_body,
        in_specs=specs,
        out_specs=[
            pl.BlockSpec((_G, 1), lambda: (0, 0)),
            pl.BlockSpec((_G, _FUSE), lambda: (0, 0)),
        ],
        out_shape=[
            jax.ShapeDtypeStruct((_G, 1), jnp.float32),
            jax.ShapeDtypeStruct((_G, _FUSE), jnp.float32),
        ],
    )(*args)


def kernel(x, edge_index, edge_attr, batch, tda, params):
    p = params
    xp = jnp.pad(x, ((0, 0), (0, 1)))
    w8 = jnp.pad(p["in_W"], ((0, 1), (0, 0)))
    h = _input_proj(xp, w8, p["in_b"].reshape(1, _H))

    src = edge_index[0]
    dst = edge_index[1]
    src1, src2 = src[:_EH1], src[_EH1:]
    dst1, dst2 = dst[:_EH1], dst[_EH1:]
    ea = _nan_clean(edge_attr, 1.0, 0.0)
    ea8 = jnp.pad(ea, ((0, 0), (0, 8 - _ED)))
    ea8_1, ea8_2 = ea8[:_EH1], ea8[_EH1:]
    zrows = jnp.zeros((_NCK + 16, _H), jnp.float32)

    for conv, ln in zip(p["convs"], p["lns"]):
        wf, ws = conv["Wf"], conv["Ws"]
        wd = jnp.concatenate([wf[:_H], ws[:_H]], axis=1)
        wsrc = jnp.concatenate([wf[_H:2 * _H], ws[_H:2 * _H]], axis=1)
        wfe8 = jnp.pad(wf[2 * _H:], ((0, 8 - _ED), (0, 0)))
        wse8 = jnp.pad(ws[2 * _H:], ((0, 8 - _ED), (0, 0)))
        d_mat, s_mat = _node_prep(h, wd, wsrc)
        u1 = _sc_gather_pair(d_mat, s_mat, dst1, src1, _EH1)
        u2 = _sc_gather_pair(d_mat, s_mat, dst2, src2, _E - _EH1)
        bf = conv["bf"].reshape(1, _H)
        bs = conv["bs"].reshape(1, _H)
        m1 = _edge_stage(u1, ea8_1, wfe8, bf, wse8, bs)
        m2 = _edge_stage(u2, ea8_2, wfe8, bf, wse8, bs)
        agg = _sc_scatter_add(m1, m2, dst1, dst2, zrows)[:_N]
        h = _update(h, agg, ln["g"].reshape(1, _H), ln["b"].reshape(1, _H))

    batch3 = batch.reshape(_N // _BN, 1, _BN)
    sums, cnt = _pool(h, batch3)

    q1w = jnp.concatenate([pk["q1W"] for pk in p["poly"]], axis=1)
    q1b = jnp.concatenate([pk["q1b"] for pk in p["poly"]], axis=0).reshape(1, -1)
    hq = _FUSE // 2
    q2w = jnp.zeros((_K * hq, _K), jnp.float32)
    for k in range(_K):
        q2w = q2w.at[k * hq:(k + 1) * hq, k].set(p["poly"][k]["q2W"][:, 0])
    q2b = jnp.concatenate([pk["q2b"] for pk in p["poly"]], axis=0).reshape(1, _K)
    lw = jnp.concatenate([pk["lW"] for pk in p["poly"]], axis=1)
    lb = jnp.concatenate([pk["lb"] for pk in p["poly"]], axis=0).reshape(1, _K)

    wpack = [
        p["out_W"], p["out_b"].reshape(1, _OUT),
        p["tda_W1"], p["tda_b1"].reshape(1, 2 * _TDA_PROJ),
        p["tda_g"].reshape(1, 2 * _TDA_PROJ), p["tda_bn"].reshape(1, 2 * _TDA_PROJ),
        p["tda_W2"], p["tda_b2"].reshape(1, _TDA_PROJ),
        p["fuse_g"].reshape(1, _FUSE), p["fuse_b"].reshape(1, _FUSE),
        p["g1W"], p["g1b"].reshape(1, _K * 4),
        p["g2W"], p["g2b"].reshape(1, _K),
        lw, lb, q1w, q1b, q2w, q2b,
    ]
    yhat2, zf = _heads(sums, cnt, tda, wpack)
    return yhat2.reshape(_G), zf


# speedup vs baseline: 1.0906x; 1.0000x over previous
import functools

import jax
import jax.numpy as jnp
import numpy as np
from jax import lax
from jax.experimental import pallas as pl
from jax.experimental.pallas import tpu as pltpu
from jax.experimental.pallas import tpu_sc as plsc

_N = 50000
_E = 800000
_G = 128
_ND = 7
_ED = 4
_H = 128
_OUT = 128
_TDA_DIM = 32
_TDA_PROJ = 64
_FUSE = _OUT + _TDA_PROJ
_K = 4
_NLAYERS = 4

_BN = 2000
_BE = 4000


def _nan_clean(v, posinf, neginf):
    return jnp.nan_to_num(v, nan=0.0, posinf=posinf, neginf=neginf)


def _in_body(x_ref, w_ref, b_ref, h_ref):
    xv = _nan_clean(x_ref[...], 3.0, -3.0)
    u = jnp.dot(xv, w_ref[...], preferred_element_type=jnp.float32) + b_ref[...]
    h_ref[...] = u * jax.nn.sigmoid(u)


def _input_proj(xp, w8, b):
    nb = _N // _BN
    return pl.pallas_call(
        _in_body,
        grid=(nb,),
        in_specs=[
            pl.BlockSpec((_BN, 8), lambda i: (i, 0)),
            pl.BlockSpec((8, _H), lambda i: (0, 0)),
            pl.BlockSpec((1, _H), lambda i: (0, 0)),
        ],
        out_specs=pl.BlockSpec((_BN, _H), lambda i: (i, 0)),
        out_shape=jax.ShapeDtypeStruct((_N, _H), jnp.float32),
    )(xp, w8, b)


def _prep_body(h_ref, wd_ref, ws_ref, d_ref, s_ref):
    h = h_ref[...]
    d_ref[...] = jnp.dot(h, wd_ref[...], preferred_element_type=jnp.float32)
    s_ref[...] = jnp.dot(h, ws_ref[...], preferred_element_type=jnp.float32)


def _node_prep(h, wd, ws):
    nb = _N // _BN
    return pl.pallas_call(
        _prep_body,
        grid=(nb,),
        in_specs=[
            pl.BlockSpec((_BN, _H), lambda i: (i, 0)),
            pl.BlockSpec((_H, 2 * _H), lambda i: (0, 0)),
            pl.BlockSpec((_H, 2 * _H), lambda i: (0, 0)),
        ],
        out_specs=[
            pl.BlockSpec((_BN, 2 * _H), lambda i: (i, 0)),
            pl.BlockSpec((_BN, 2 * _H), lambda i: (i, 0)),
        ],
        out_shape=[
            jax.ShapeDtypeStruct((_N, 2 * _H), jnp.float32),
            jax.ShapeDtypeStruct((_N, 2 * _H), jnp.float32),
        ],
    )(h, wd, ws)


_NW = 32
_EH1 = 384000
_BG = 40
_SBG = 1000
_RING = 4


def _sc_gather_pair(d_mat, s_mat, dst, src, n_edges):
    epw = n_edges // _NW
    mesh = plsc.VectorSubcoreMesh(core_axis_name="c", subcore_axis_name="s")

    @functools.partial(
        pl.kernel,
        mesh=mesh,
        out_type=jax.ShapeDtypeStruct((n_edges, 2 * _H), jnp.float32),
        scratch_types=[
            pltpu.VMEM((_SBG,), jnp.int32),
            pltpu.VMEM((_SBG,), jnp.int32),
            pltpu.VMEM((_RING, _BG, 2 * _H), jnp.float32),
            pltpu.VMEM((_RING, _BG, 2 * _H), jnp.float32),
            pltpu.SemaphoreType.DMA,
            pltpu.SemaphoreType.DMA,
            pltpu.SemaphoreType.DMA,
            pltpu.SemaphoreType.DMA,
        ],
    )
    def k(d_hbm, s_hbm, dst_hbm, src_hbm, u_hbm,
          idxd, idxs, dbufs, sbufs, gsd, gss, wsd, wss):
        wid = lax.axis_index("s") * 2 + lax.axis_index("c")
        base = wid * epw

        def super_body(sb, carry):
            soff = base + sb * _SBG
            pltpu.sync_copy(dst_hbm.at[pl.ds(soff, _SBG)], idxd)
            pltpu.sync_copy(src_hbm.at[pl.ds(soff, _SBG)], idxs)
            g = [None] * _RING
            w = [None] * _RING

            def issue_gather(j):
                r = j % _RING
                c1 = pltpu.async_copy(
                    d_hbm.at[idxd.at[pl.ds(j * _BG, _BG)]], dbufs.at[r], gsd)
                c2 = pltpu.async_copy(
                    s_hbm.at[idxs.at[pl.ds(j * _BG, _BG)]], sbufs.at[r], gss)
                return (c1, c2)

            def issue_wb(j):
                r = j % _RING

                def add_row(i, carry):
                    for v in range(2 * _H // 16):
                        sl = pl.ds(v * 16, 16)
                        dbufs[r, i, sl] = dbufs[r, i, sl] + sbufs[r, i, sl]
                    return carry

                g[r][0].wait()
                g[r][1].wait()
                lax.fori_loop(0, _BG, add_row, 0)
                off = soff + j * _BG
                c1 = pltpu.async_copy(dbufs.at[r], u_hbm.at[pl.ds(off, _BG)], wsd)
                return (c1,)

            nblk = _SBG // _BG
            for j in range(nblk):
                r = j % _RING
                if w[r] is not None:
                    w[r][0].wait()
                    w[r] = None
                g[r] = issue_gather(j)
                jw = j - (_RING - 1)
                if jw >= 0:
                    w[jw % _RING] = issue_wb(jw)
            for jw in range(nblk - (_RING - 1), nblk):
                w[jw % _RING] = issue_wb(jw)
            for r in range(_RING):
                if w[r] is not None:
                    w[r][0].wait()
            return carry

        lax.fori_loop(0, epw // _SBG, super_body, 0)

    return k(d_mat, s_mat, dst, src)


_NCK = 12544
_NPAD = 4 * _NCK
_BS = 80
_RPT = _NCK // 16


_SBS = 2000


def _sc_scatter_add(m1, m2, dst1, dst2, zrows):
    n1, n2 = m1.shape[0], m2.shape[0]
    mesh = plsc.VectorSubcoreMesh(core_axis_name="c", subcore_axis_name="s")

    @functools.partial(
        pl.kernel,
        mesh=mesh,
        out_type=jax.ShapeDtypeStruct((_NPAD, _H), jnp.float32),
        scratch_types=[
            pltpu.VMEM((_SBS,), jnp.int32),
            pltpu.VMEM((_BS,), jnp.int32),
            pltpu.VMEM((2, _BS, _H), jnp.float32),
            pltpu.VMEM_SHARED((_NCK + 16, _H), jnp.float32),
            pltpu.SemaphoreType.DMA,
        ],
    )
    def k(m1_hbm, m2_hbm, dst1_hbm, dst2_hbm, z_hbm, agg_hbm,
          dstb, idxb, mbufs, shard, lsem):
        cid = lax.axis_index("c")
        sid = lax.axis_index("s")
        nblk = _SBS // _BS
        for rnd in range(2):
            chunk = rnd * 2 + cid
            nbase = chunk * _NCK
            dummy = _NCK + sid

            @pl.when(sid == 0)
            def _():
                pltpu.sync_copy(z_hbm, shard)

            plsc.subcore_barrier()

            for m_hbm, dst_hbm, n_e in ((m1_hbm, dst1_hbm, n1),
                                        (m2_hbm, dst2_hbm, n2)):
                ept = n_e // 16
                ebase = sid * ept

                def super_body(sb, carry):
                    soff = ebase + sb * _SBS
                    pltpu.sync_copy(dst_hbm.at[pl.ds(soff, _SBS)], dstb)
                    L = [None, None]
                    L[0] = pltpu.async_copy(
                        m_hbm.at[pl.ds(soff, _BS)], mbufs.at[0], lsem)
                    L[1] = pltpu.async_copy(
                        m_hbm.at[pl.ds(soff + _BS, _BS)], mbufs.at[1], lsem)
                    for j in range(nblk):
                        r = j % 2
                        L[r].wait()
                        for v in range(_BS // 16):
                            d = dstb[pl.ds(j * _BS + v * 16, 16)]
                            loc = d - nbase
                            ok = (loc >= 0) & (loc < _NCK)
                            idxb[pl.ds(v * 16, 16)] = jnp.where(ok, loc, dummy)
                        pltpu.sync_copy(mbufs.at[r], shard.at[idxb], add=True)
                        if j + 2 < nblk:
                            L[r] = pltpu.async_copy(
                                m_hbm.at[pl.ds(soff + (j + 2) * _BS, _BS)],
                                mbufs.at[r], lsem)
                    return carry

                lax.fori_loop(0, ept // _SBS, super_body, 0)
            plsc.subcore_barrier()
            pltpu.sync_copy(shard.at[pl.ds(sid * _RPT, _RPT)],
                            agg_hbm.at[pl.ds(nbase + sid * _RPT, _RPT)])
            plsc.subcore_barrier()

    return k(m1, m2, dst1, dst2, zrows)


def _edge_body(u_ref, ea_ref, wfe_ref, bf_ref, wse_ref, bs_ref, m_ref):
    u = u_ref[...]
    ea = ea_ref[...]
    uf = (u[:, :_H]
          + jnp.dot(ea, wfe_ref[...], preferred_element_type=jnp.float32)
          + bf_ref[...])
    us = (u[:, _H:]
          + jnp.dot(ea, wse_ref[...], preferred_element_type=jnp.float32)
          + bs_ref[...])
    sig = jax.nn.sigmoid(uf)
    sp = jnp.maximum(us, 0.0) + jnp.log1p(jnp.exp(-jnp.abs(us)))
    m_ref[...] = sig * sp


def _edge_stage(u, ea8, wfe8, bf, wse8, bs):
    nb = u.shape[0] // _BE
    return pl.pallas_call(
        _edge_body,
        grid=(nb,),
        in_specs=[
            pl.BlockSpec((_BE, 2 * _H), lambda i: (i, 0)),
            pl.BlockSpec((_BE, 8), lambda i: (i, 0)),
            pl.BlockSpec((8, _H), lambda i: (0, 0)),
            pl.BlockSpec((1, _H), lambda i: (0, 0)),
            pl.BlockSpec((8, _H), lambda i: (0, 0)),
            pl.BlockSpec((1, _H), lambda i: (0, 0)),
        ],
        out_specs=pl.BlockSpec((_BE, _H), lambda i: (i, 0)),
        out_shape=jax.ShapeDtypeStruct((u.shape[0], _H), jnp.float32),
    )(u, ea8, wfe8, bf, wse8, bs)


def _upd_body(h_ref, agg_ref, g_ref, b_ref, out_ref):
    h = h_ref[...]
    c = agg_ref[...] + h
    y = c * jax.nn.sigmoid(c) + h
    m = jnp.mean(y, axis=-1, keepdims=True)
    v = jnp.mean((y - m) ** 2, axis=-1, keepdims=True)
    out_ref[...] = (y - m) * jax.lax.rsqrt(v + 1e-5) * g_ref[...] + b_ref[...]


def _update(h, agg, g, b):
    nb = _N // _BN
    return pl.pallas_call(
        _upd_body,
        grid=(nb,),
        in_specs=[
            pl.BlockSpec((_BN, _H), lambda i: (i, 0)),
            pl.BlockSpec((_BN, _H), lambda i: (i, 0)),
            pl.BlockSpec((1, _H), lambda i: (0, 0)),
            pl.BlockSpec((1, _H), lambda i: (0, 0)),
        ],
        out_specs=pl.BlockSpec((_BN, _H), lambda i: (i, 0)),
        out_shape=jax.ShapeDtypeStruct((_N, _H), jnp.float32),
    )(h, agg, g, b)


def _pool_body(h_ref, batch_ref, sums_ref, cnt_ref):
    i = pl.program_id(0)

    @pl.when(i == 0)
    def _():
        sums_ref[...] = jnp.zeros_like(sums_ref)
        cnt_ref[...] = jnp.zeros_like(cnt_ref)

    b = batch_ref[0, 0, :]
    onehot = (b[:, None] == jax.lax.broadcasted_iota(jnp.int32, (_BN, _G), 1))
    onehot = onehot.astype(jnp.float32)
    sums_ref[...] += jax.lax.dot_general(
        onehot, h_ref[...], (((0,), (0,)), ((), ())),
        preferred_element_type=jnp.float32)
    cnt_ref[...] += jnp.sum(onehot, axis=0, keepdims=True)


def _pool(h, batch3):
    nb = _N // _BN
    return pl.pallas_call(
        _pool_body,
        grid=(nb,),
        in_specs=[
            pl.BlockSpec((_BN, _H), lambda i: (i, 0)),
            pl.BlockSpec((1, 1, _BN), lambda i: (i, 0, 0)),
        ],
        out_specs=[
            pl.BlockSpec((_G, _H), lambda i: (0, 0)),
            pl.BlockSpec((1, _G), lambda i: (0, 0)),
        ],
        out_shape=[
            jax.ShapeDtypeStruct((_G, _H), jnp.float32),
            jax.ShapeDtypeStruct((1, _G), jnp.float32),
        ],
    )(h, batch3)


def _ln_rows(x, g, b):
    m = jnp.mean(x, axis=-1, keepdims=True)
    v = jnp.mean((x - m) ** 2, axis=-1, keepdims=True)
    return (x - m) * jax.lax.rsqrt(v + 1e-5) * g + b


def _head_body(sums_ref, cnt_ref, tda_ref,
               outw_ref, outb_ref, tw1_ref, tb1_ref, tg_ref, tbn_ref,
               tw2_ref, tb2_ref, fg_ref, fb_ref,
               g1w_ref, g1b_ref, g2w_ref, g2b_ref,
               lw_ref, lb_ref, q1w_ref, q1b_ref, q2w_ref, q2b_ref,
               yhat_ref, zf_ref):
    cnt = jnp.maximum(cnt_ref[...], 1.0)
    pooled = sums_ref[...] / cnt.reshape(_G, 1)
    z_gnn = jnp.dot(pooled, outw_ref[...],
                    preferred_element_type=jnp.float32) + outb_ref[...]
    tda = _nan_clean(tda_ref[...], 3.0, -3.0)
    t = jnp.dot(tda, tw1_ref[...], preferred_element_type=jnp.float32) + tb1_ref[...]
    t = t * jax.nn.sigmoid(t)
    t = _ln_rows(t, tg_ref[...], tbn_ref[...])
    z_tda = jnp.dot(t, tw2_ref[...], preferred_element_type=jnp.float32) + tb2_ref[...]
    z_gnn = jnp.nan_to_num(z_gnn, nan=0.0)
    z_tda = jnp.nan_to_num(z_tda, nan=0.0)
    zf = jnp.concatenate([z_gnn, z_tda], axis=-1)
    zf = _ln_rows(zf, fg_ref[...], fb_ref[...])
    zf_ref[...] = zf
    gh = jnp.dot(zf, g1w_ref[...], preferred_element_type=jnp.float32) + g1b_ref[...]
    gh = gh * jax.nn.sigmoid(gh)
    glog = jnp.dot(gh, g2w_ref[...], preferred_element_type=jnp.float32) + g2b_ref[...]
    gates = jax.nn.softmax(glog, axis=-1)
    lin = jnp.dot(zf, lw_ref[...], preferred_element_type=jnp.float32) + lb_ref[...]
    q = jnp.dot(zf, q1w_ref[...], preferred_element_type=jnp.float32) + q1b_ref[...]
    q = q * jax.nn.sigmoid(q)
    quad = jnp.dot(q, q2w_ref[...], preferred_element_type=jnp.float32) + q2b_ref[...]
    preds = lin + quad
    yhat_ref[...] = jnp.sum(gates * preds, axis=-1, keepdims=True)


def _heads(sums, cnt, tda, wpack):
    specs = [
        pl.BlockSpec((_G, _H), lambda: (0, 0)),
        pl.BlockSpec((1, _G), lambda: (0, 0)),
        pl.BlockSpec((_G, _TDA_DIM), lambda: (0, 0)),
    ]
    args = [sums, cnt, tda]
    for w in wpack:
        args.append(w)
        specs.append(pl.BlockSpec(w.shape, lambda: (0,) * w.ndim))
    return pl.pallas_call(
        _head_body,
        in_specs=specs,
        out_specs=[
            pl.BlockSpec((_G, 1), lambda: (0, 0)),
            pl.BlockSpec((_G, _FUSE), lambda: (0, 0)),
        ],
        out_shape=[
            jax.ShapeDtypeStruct((_G, 1), jnp.float32),
            jax.ShapeDtypeStruct((_G, _FUSE), jnp.float32),
        ],
    )(*args)


def kernel(x, edge_index, edge_attr, batch, tda, params):
    p = params
    xp = jnp.pad(x, ((0, 0), (0, 1)))
    w8 = jnp.pad(p["in_W"], ((0, 1), (0, 0)))
    h = _input_proj(xp, w8, p["in_b"].reshape(1, _H))

    src = edge_index[0]
    dst = edge_index[1]
    src1, src2 = src[:_EH1], src[_EH1:]
    dst1, dst2 = dst[:_EH1], dst[_EH1:]
    ea = _nan_clean(edge_attr, 1.0, 0.0)
    ea8 = jnp.pad(ea, ((0, 0), (0, 8 - _ED)))
    ea8_1, ea8_2 = ea8[:_EH1], ea8[_EH1:]
    zrows = jnp.zeros((_NCK + 16, _H), jnp.float32)

    for conv, ln in zip(p["convs"], p["lns"]):
        wf, ws = conv["Wf"], conv["Ws"]
        wd = jnp.concatenate([wf[:_H], ws[:_H]], axis=1)
        wsrc = jnp.concatenate([wf[_H:2 * _H], ws[_H:2 * _H]], axis=1)
        wfe8 = jnp.pad(wf[2 * _H:], ((0, 8 - _ED), (0, 0)))
        wse8 = jnp.pad(ws[2 * _H:], ((0, 8 - _ED), (0, 0)))
        d_mat, s_mat = _node_prep(h, wd, wsrc)
        u1 = _sc_gather_pair(d_mat, s_mat, dst1, src1, _EH1)
        u2 = _sc_gather_pair(d_mat, s_mat, dst2, src2, _E - _EH1)
        bf = conv["bf"].reshape(1, _H)
        bs = conv["bs"].reshape(1, _H)
        m1 = _edge_stage(u1, ea8_1, wfe8, bf, wse8, bs)
        m2 = _edge_stage(u2, ea8_2, wfe8, bf, wse8, bs)
        agg = _sc_scatter_add(m1, m2, dst1, dst2, zrows)[:_N]
        h = _update(h, agg, ln["g"].reshape(1, _H), ln["b"].reshape(1, _H))

    batch3 = batch.reshape(_N // _BN, 1, _BN)
    sums, cnt = _pool(h, batch3)

    q1w = jnp.concatenate([pk["q1W"] for pk in p["poly"]], axis=1)
    q1b = jnp.concatenate([pk["q1b"] for pk in p["poly"]], axis=0).reshape(1, -1)
    hq = _FUSE // 2
    q2w = jnp.zeros((_K * hq, _K), jnp.float32)
    for k in range(_K):
        q2w = q2w.at[k * hq:(k + 1) * hq, k].set(p["poly"][k]["q2W"][:, 0])
    q2b = jnp.concatenate([pk["q2b"] for pk in p["poly"]], axis=0).reshape(1, _K)
    lw = jnp.concatenate([pk["lW"] for pk in p["poly"]], axis=1)
    lb = jnp.concatenate([pk["lb"] for pk in p["poly"]], axis=0).reshape(1, _K)

    wpack = [
        p["out_W"], p["out_b"].reshape(1, _OUT),
        p["tda_W1"], p["tda_b1"].reshape(1, 2 * _TDA_PROJ),
        p["tda_g"].reshape(1, 2 * _TDA_PROJ), p["tda_bn"].reshape(1, 2 * _TDA_PROJ),
        p["tda_W2"], p["tda_b2"].reshape(1, _TDA_PROJ),
        p["fuse_g"].reshape(1, _FUSE), p["fuse_b"].reshape(1, _FUSE),
        p["g1W"], p["g1b"].reshape(1, _K * 4),
        p["g2W"], p["g2b"].reshape(1, _K),
        lw, lb, q1w, q1b, q2w, q2b,
    ]
    yhat2, zf = _heads(sums, cnt, tda, wpack)
    return yhat2.reshape(_G), zf
